# baseline (speedup 1.0000x reference)
import functools

import jax
import jax.numpy as jnp
from jax import lax
from jax.experimental import pallas as pl
from jax.experimental.pallas import tpu as pltpu
from jax.experimental.pallas import tpu_sc as plsc

_D = 64
_L = 16
_W = 128

_mesh = plsc.VectorSubcoreMesh(
    core_axis_name="c", subcore_axis_name="s", num_cores=1)


def _bcast_lane(vec, lane):
    return vec.at[jnp.full((_L,), lane, jnp.int32)].get(
        mode="promise_in_bounds")


@functools.partial(
    pl.kernel,
    mesh=_mesh,
    out_type=jax.ShapeDtypeStruct((2 * _L,), jnp.float32),
    scratch_types=[
        pltpu.VMEM((_L,), jnp.int32),
        pltpu.VMEM((_L,), jnp.int32),
        pltpu.VMEM((_D, _W), jnp.float32),
        pltpu.VMEM((_D, _W), jnp.float32),
        pltpu.VMEM((1, _D * _D), jnp.float32),
        pltpu.VMEM((_D // _L, _L), jnp.float32),
        pltpu.VMEM((_L,), jnp.float32),
        pltpu.SemaphoreType.DMA,
        pltpu.SemaphoreType.DMA,
        pltpu.SemaphoreType.DMA,
    ],
)
def _sc_scores(idx_hbm, et_hbm, r_hbm, out_hbm,
               idx_v, pidx_v, blk_s, blk_o, r_row, obuf, out_v,
               sem_s, sem_o, sem_r):
    t = lax.axis_index("s")

    @pl.when(t < 2)
    def _():
        pltpu.sync_copy(idx_hbm.at[0, t], idx_v)
        row = idx_v[...]
        pidx_v[...] = _bcast_lane(row, jnp.int32(2))
        cp_r = pltpu.async_copy(r_hbm.at[pidx_v.at[pl.ds(0, 1)]], r_row, sem_r)
        s = row[0]
        o = row[1]
        s_col = pl.multiple_of((s // _W) * _W, _W)
        o_col = pl.multiple_of((o // _W) * _W, _W)
        cp_s = pltpu.async_copy(et_hbm.at[:, pl.ds(s_col, _W)], blk_s, sem_s)
        cp_o = pltpu.async_copy(et_hbm.at[:, pl.ds(o_col, _W)], blk_o, sem_o)
        cp_o.wait()

        s_sub = s - s_col
        o_sub = o - o_col
        s_base = pl.multiple_of((s_sub // _L) * _L, _L)
        o_base = pl.multiple_of((o_sub // _L) * _L, _L)
        s_lane = s_sub - s_base
        o_lane = o_sub - o_base

        lanes = lax.iota(jnp.int32, _L)

        zero = jnp.zeros((_L,), jnp.float32)
        for k in range(_D // _L):
            obuf[k, :] = zero

        def o_body(j, carry):
            r = blk_o[j, pl.ds(o_base, _L)]
            b = _bcast_lane(r, o_lane)
            plsc.addupdate(
                obuf.at[j // _L], jnp.where(lanes == j % _L, b, 0.0))
            return carry

        lax.fori_loop(0, _D, o_body, 0)
        o_vecs = [obuf[k, :] for k in range(_D // _L)]

        cp_s.wait()
        cp_r.wait()

        def fma_body(i, accs):
            s_i = _bcast_lane(blk_s[i, pl.ds(s_base, _L)], s_lane)
            return tuple(
                accs[k] + s_i * r_row[0, pl.ds(i * _D + k * _L, _L)] * o_vecs[k]
                for k in range(_D // _L))

        acc = lax.fori_loop(
            0, _D, fma_body,
            tuple(jnp.zeros((_L,), jnp.float32) for _ in range(_D // _L)))
        total = (acc[0] + acc[1]) + (acc[2] + acc[3])
        for step in (8, 4, 2, 1):
            total = total + total.at[lanes ^ step].get(
                mode="promise_in_bounds")
        out_v[...] = total
        pltpu.sync_copy(out_v, out_hbm.at[pl.ds(t * _L, _L)])


def kernel(x, E, R):
    idx = jnp.pad(x.astype(jnp.int32), ((0, 0), (0, 0), (0, _L - 3)))
    return _sc_scores(idx, E.T, R)[::_L]


# speedup vs baseline: 1.0072x; 1.0072x over previous
import functools

import jax
import jax.numpy as jnp
from jax import lax
from jax.experimental import pallas as pl
from jax.experimental.pallas import tpu as pltpu
from jax.experimental.pallas import tpu_sc as plsc

_D = 64
_L = 16
_W = 128

_mesh = plsc.VectorSubcoreMesh(
    core_axis_name="c", subcore_axis_name="s", num_cores=1)


def _bcast_lane(vec, lane):
    return vec.at[jnp.full((_L,), lane, jnp.int32)].get(
        mode="promise_in_bounds")


@functools.partial(
    pl.kernel,
    mesh=_mesh,
    out_type=jax.ShapeDtypeStruct((2 * _L,), jnp.float32),
    scratch_types=[
        pltpu.VMEM((_L,), jnp.int32),
        pltpu.VMEM((_L,), jnp.int32),
        pltpu.VMEM((_D, _W), jnp.float32),
        pltpu.VMEM((_D, _W), jnp.float32),
        pltpu.VMEM((1, _D * _D), jnp.float32),
        pltpu.VMEM((_D // _L, _L), jnp.float32),
        pltpu.VMEM((_L,), jnp.float32),
        pltpu.SemaphoreType.DMA,
        pltpu.SemaphoreType.DMA,
        pltpu.SemaphoreType.DMA,
    ],
)
def _sc_scores(idx_hbm, et_hbm, r_hbm, out_hbm,
               idx_v, pidx_v, blk_s, blk_o, r_row, obuf, out_v,
               sem_s, sem_o, sem_r):
    t = lax.axis_index("s")

    @pl.when(t < 2)
    def _():
        pltpu.sync_copy(idx_hbm.at[0, t], idx_v)
        row = idx_v[...]
        o = row[1]
        o_col = pl.multiple_of((o // _W) * _W, _W)
        cp_o = pltpu.async_copy(et_hbm.at[:, pl.ds(o_col, _W)], blk_o, sem_o)
        s = row[0]
        s_col = pl.multiple_of((s // _W) * _W, _W)
        cp_s = pltpu.async_copy(et_hbm.at[:, pl.ds(s_col, _W)], blk_s, sem_s)
        pidx_v[...] = _bcast_lane(row, jnp.int32(2))
        cp_r = pltpu.async_copy(r_hbm.at[pidx_v.at[pl.ds(0, 1)]], r_row, sem_r)
        cp_o.wait()

        s_sub = s - s_col
        o_sub = o - o_col
        s_base = pl.multiple_of((s_sub // _L) * _L, _L)
        o_base = pl.multiple_of((o_sub // _L) * _L, _L)
        s_lane = s_sub - s_base
        o_lane = o_sub - o_base

        lanes = lax.iota(jnp.int32, _L)

        zero = jnp.zeros((_L,), jnp.float32)
        for k in range(_D // _L):
            obuf[k, :] = zero

        def o_body(j, carry):
            r = blk_o[j, pl.ds(o_base, _L)]
            b = _bcast_lane(r, o_lane)
            plsc.addupdate(
                obuf.at[j // _L], jnp.where(lanes == j % _L, b, 0.0))
            return carry

        lax.fori_loop(0, _D, o_body, 0)
        o_vecs = [obuf[k, :] for k in range(_D // _L)]

        cp_s.wait()
        cp_r.wait()

        def fma_body(i, accs):
            s_i = _bcast_lane(blk_s[i, pl.ds(s_base, _L)], s_lane)
            return tuple(
                accs[k] + s_i * r_row[0, pl.ds(i * _D + k * _L, _L)] * o_vecs[k]
                for k in range(_D // _L))

        acc = lax.fori_loop(
            0, _D, fma_body,
            tuple(jnp.zeros((_L,), jnp.float32) for _ in range(_D // _L)))
        total = (acc[0] + acc[1]) + (acc[2] + acc[3])
        for step in (8, 4, 2, 1):
            total = total + total.at[lanes ^ step].get(
                mode="promise_in_bounds")
        out_v[...] = total
        pltpu.sync_copy(out_v, out_hbm.at[pl.ds(t * _L, _L)])


def kernel(x, E, R):
    idx = jnp.pad(x.astype(jnp.int32), ((0, 0), (0, 0), (0, _L - 3)))
    return _sc_scores(idx, E.T, R)[::_L]


# speedup vs baseline: 1.0153x; 1.0080x over previous
import functools

import jax
import jax.numpy as jnp
from jax import lax
from jax.experimental import pallas as pl
from jax.experimental.pallas import tpu as pltpu
from jax.experimental.pallas import tpu_sc as plsc

_D = 64
_L = 16
_W = 128

_mesh = plsc.VectorSubcoreMesh(
    core_axis_name="c", subcore_axis_name="s", num_cores=1)


def _bcast_lane(vec, lane):
    return vec.at[jnp.full((_L,), lane, jnp.int32)].get(
        mode="promise_in_bounds")


@functools.partial(
    pl.kernel,
    mesh=_mesh,
    out_type=jax.ShapeDtypeStruct((2 * _L,), jnp.float32),
    scratch_types=[
        pltpu.VMEM((_L,), jnp.int32),
        pltpu.VMEM((_L,), jnp.int32),
        pltpu.VMEM((_D, _W), jnp.float32),
        pltpu.VMEM((_D, _W), jnp.float32),
        pltpu.VMEM((1, _D * _D), jnp.float32),
        pltpu.VMEM((_D // _L, _L), jnp.float32),
        pltpu.VMEM((_L,), jnp.float32),
        pltpu.SemaphoreType.DMA,
        pltpu.SemaphoreType.DMA,
        pltpu.SemaphoreType.DMA,
    ],
)
def _sc_scores(idx_hbm, et_hbm, r_hbm, out_hbm,
               idx_v, pidx_v, blk_s, blk_o, r_row, obuf, out_v,
               sem_s, sem_o, sem_r):
    t = lax.axis_index("s")

    @pl.when(t < 2)
    def _():
        pltpu.sync_copy(idx_hbm, idx_v.at[pl.ds(0, 6)])
        row = idx_v[...]
        is0 = t == 0
        o = jnp.where(is0, row[1], row[4])
        o_col = pl.multiple_of((o // _W) * _W, _W)
        cp_o = pltpu.async_copy(et_hbm.at[:, pl.ds(o_col, _W)], blk_o, sem_o)
        s = jnp.where(is0, row[0], row[3])
        s_col = pl.multiple_of((s // _W) * _W, _W)
        cp_s = pltpu.async_copy(et_hbm.at[:, pl.ds(s_col, _W)], blk_s, sem_s)
        pidx_v[...] = jnp.where(
            is0,
            _bcast_lane(row, jnp.int32(2)),
            _bcast_lane(row, jnp.int32(5)))
        cp_r = pltpu.async_copy(r_hbm.at[pidx_v.at[pl.ds(0, 1)]], r_row, sem_r)
        cp_o.wait()

        s_sub = s - s_col
        o_sub = o - o_col
        s_base = pl.multiple_of((s_sub // _L) * _L, _L)
        o_base = pl.multiple_of((o_sub // _L) * _L, _L)
        s_lane = s_sub - s_base
        o_lane = o_sub - o_base

        lanes = lax.iota(jnp.int32, _L)

        zero = jnp.zeros((_L,), jnp.float32)
        for k in range(_D // _L):
            obuf[k, :] = zero

        def o_body(j, carry):
            r = blk_o[j, pl.ds(o_base, _L)]
            b = _bcast_lane(r, o_lane)
            plsc.addupdate(
                obuf.at[j // _L], jnp.where(lanes == j % _L, b, 0.0))
            return carry

        lax.fori_loop(0, _D, o_body, 0)
        o_vecs = [obuf[k, :] for k in range(_D // _L)]

        cp_s.wait()
        cp_r.wait()

        def fma_body(i, accs):
            s_i = _bcast_lane(blk_s[i, pl.ds(s_base, _L)], s_lane)
            return tuple(
                accs[k] + s_i * r_row[0, pl.ds(i * _D + k * _L, _L)] * o_vecs[k]
                for k in range(_D // _L))

        acc = lax.fori_loop(
            0, _D, fma_body,
            tuple(jnp.zeros((_L,), jnp.float32) for _ in range(_D // _L)))
        total = (acc[0] + acc[1]) + (acc[2] + acc[3])
        for step in (8, 4, 2, 1):
            total = total + total.at[lanes ^ step].get(
                mode="promise_in_bounds")
        out_v[...] = total
        pltpu.sync_copy(out_v, out_hbm.at[pl.ds(t * _L, _L)])


def kernel(x, E, R):
    return _sc_scores(x.reshape(6).astype(jnp.int32), E.T, R)[::_L]
